# Initial kernel scaffold; baseline (speedup 1.0000x reference)
import jax
import jax.numpy as jnp
from jax.experimental import pallas as pl


def kernel(x, edge_index, W1_rel, b1_rel, W1_root, W2_rel, b2_rel, W2_root, W3_rel, b3_rel, W3_root, W_pos, b_pos, W_rot, b_rot):
    raise NotImplementedError("write your pallas kernel here")


# speedup vs baseline: 3.5194x; 3.5194x over previous
import functools

import jax
import jax.numpy as jnp
from jax import lax
from jax.experimental import pallas as pl
from jax.experimental.pallas import tpu as pltpu
from jax.experimental.pallas import tpu_sc as plsc

M_TILE = 1000

_NS = 16
_NC = 2
_K = 128


def _leaky(x):
    return jnp.where(x > 0, x, 0.01 * x)


def _mm_first_body(x_ref, w_ref, o_ref):
    o_ref[0] = jnp.dot(x_ref[...], w_ref[0], preferred_element_type=jnp.float32)


def _mm_first(x, w_cat, w_out):
    n, d_in = x.shape
    nb = w_cat.shape[1] // w_out
    nm = n // M_TILE
    w_blk = jnp.moveaxis(w_cat.reshape(d_in, nb, w_out), 1, 0)
    return pl.pallas_call(
        _mm_first_body,
        grid=(nm, nb),
        in_specs=[
            pl.BlockSpec((M_TILE, d_in), lambda m, b: (m, 0)),
            pl.BlockSpec((1, d_in, w_out), lambda m, b: (b, 0, 0)),
        ],
        out_specs=pl.BlockSpec((1, M_TILE, w_out), lambda m, b: (b, m, 0)),
        out_shape=jax.ShapeDtypeStruct((nb, n, w_out), jnp.float32),
    )(x, w_blk)


def _hidden(agg_ref, r_ref, b_ref, partial, cols=None):
    nb = agg_ref.shape[0]
    if partial:
        agg = agg_ref[0] + agg_ref[1]
        r = r_ref[0]
        if cols is not None:
            agg = agg[:, :cols]
            r = r[:, cols:]
    else:
        agg = jnp.concatenate([agg_ref[i] for i in range(nb)], axis=1)
        r = jnp.concatenate([r_ref[i] for i in range(nb)], axis=1)
    return _leaky(agg + r + b_ref[...])


def _mm_mid_body(agg_ref, r_ref, b_ref, w_ref, o_ref, *, partial, cols):
    h = _hidden(agg_ref, r_ref, b_ref, partial, cols)
    o_ref[0] = jnp.dot(h, w_ref[0], preferred_element_type=jnp.float32)


def _mm_mid(agg, r, bias, w_cat, w_out, partial=False, cols=None):
    nb_in, _, w_in = agg.shape
    n = r.shape[1]
    d_in = bias.shape[0]
    nb = w_cat.shape[1] // w_out
    nm = n // M_TILE
    w_blk = jnp.moveaxis(w_cat.reshape(d_in, nb, w_out), 1, 0)
    body = functools.partial(_mm_mid_body, partial=partial, cols=cols)
    return pl.pallas_call(
        body,
        grid=(nm, nb),
        in_specs=[
            pl.BlockSpec((nb_in, M_TILE, w_in), lambda m, b: (0, m, 0)),
            pl.BlockSpec((r.shape[0], M_TILE, w_in), lambda m, b: (0, m, 0)),
            pl.BlockSpec((1, d_in), lambda m, b: (0, 0)),
            pl.BlockSpec((1, d_in, w_out), lambda m, b: (b, 0, 0)),
        ],
        out_specs=pl.BlockSpec((1, M_TILE, w_out), lambda m, b: (b, m, 0)),
        out_shape=jax.ShapeDtypeStruct((nb, n, w_out), jnp.float32),
    )(agg, r, bias.reshape(1, d_in), w_blk)


def _head_body(agg_ref, r_ref, b_ref, w_ref, bh_ref, o_ref, *, cols):
    h = _hidden(agg_ref, r_ref, b_ref, True, cols)
    y = jnp.dot(h, w_ref[...], preferred_element_type=jnp.float32) + bh_ref[...]
    pos = y[:, 0:3]
    rot = y[:, 3:7]
    norm = jnp.maximum(
        jnp.sqrt(jnp.sum(rot * rot, axis=1, keepdims=True)), 1e-12)
    o_ref[...] = jnp.concatenate(
        [pos, rot / norm, jnp.zeros_like(y[:, 7:8])], axis=1)


def _head(agg, r, bias, w_head, b_head, cols):
    w_in = agg.shape[2]
    n = r.shape[1]
    d_in = bias.shape[0]
    w_head = jnp.pad(w_head, ((0, 0), (0, 1)))
    b_head = jnp.pad(b_head, (0, 1))
    d_out = w_head.shape[1]
    nm = n // M_TILE
    body = functools.partial(_head_body, cols=cols)
    return pl.pallas_call(
        body,
        grid=(nm,),
        in_specs=[
            pl.BlockSpec((2, M_TILE, w_in), lambda m: (0, m, 0)),
            pl.BlockSpec((1, M_TILE, w_in), lambda m: (0, m, 0)),
            pl.BlockSpec((1, d_in), lambda m: (0, 0)),
            pl.BlockSpec((d_in, d_out), lambda m: (0, 0)),
            pl.BlockSpec((1, d_out), lambda m: (0, 0)),
        ],
        out_specs=pl.BlockSpec((M_TILE, d_out), lambda m: (m, 0)),
        out_shape=jax.ShapeDtypeStruct((n, d_out), jnp.float32),
    )(agg, r, bias.reshape(1, d_in), w_head, b_head.reshape(1, d_out))


def _sc_segsum_call(p_blocked, idx, zeros, n_pad, esplit, c_round, n_tasks):
    bpc = n_tasks // _NC
    stripe = n_pad // _NS
    w = p_blocked.shape[2]

    def body(p_ref, idx_ref, zeros_ref, out_ref,
             ibuf0, ibuf1, rbuf0, rbuf1, acc,
             semi0, semi1, semg0, semg1):
        c = lax.axis_index("c")
        s = lax.axis_index("s")
        row0 = s * stripe
        for b in range(bpc):
            t = c * bpc + b
            blk = 0 if esplit == 2 else t
            base = t * c_round if esplit == 2 else 0
            pltpu.sync_copy(zeros_ref, acc.at[pl.ds(row0, stripe)])
            plsc.subcore_barrier()

            pltpu.sync_copy(idx_ref.at[s, base], ibuf0)
            pltpu.async_copy(p_ref.at[blk].at[ibuf0.at[0]], rbuf0, semg0)
            pltpu.async_copy(idx_ref.at[s, base + 1], ibuf1, semi1)

            def pair(i, _):
                k0 = base + 2 * i
                pltpu.make_async_copy(
                    idx_ref.at[s, k0 + 1], ibuf1, semi1).wait()
                pltpu.make_async_copy(
                    p_ref.at[blk].at[ibuf0.at[0]], rbuf0, semg0).wait()
                pltpu.async_copy(p_ref.at[blk].at[ibuf1.at[0]], rbuf1, semg1)
                pltpu.sync_copy(rbuf0, acc.at[ibuf0.at[1]], add=True)
                pltpu.async_copy(idx_ref.at[s, k0 + 2], ibuf0, semi0)
                pltpu.make_async_copy(
                    idx_ref.at[s, k0 + 2], ibuf0, semi0).wait()
                pltpu.make_async_copy(
                    p_ref.at[blk].at[ibuf1.at[0]], rbuf1, semg1).wait()
                pltpu.async_copy(p_ref.at[blk].at[ibuf0.at[0]], rbuf0, semg0)
                pltpu.sync_copy(rbuf1, acc.at[ibuf1.at[1]], add=True)
                pltpu.async_copy(idx_ref.at[s, k0 + 3], ibuf1, semi1)
                return 0

            lax.fori_loop(0, c_round // 2, pair, 0)
            pltpu.make_async_copy(
                p_ref.at[blk].at[ibuf0.at[0]], rbuf0, semg0).wait()
            pltpu.make_async_copy(
                idx_ref.at[s, base + c_round + 1], ibuf1, semi1).wait()
            plsc.subcore_barrier()
            pltpu.sync_copy(acc.at[pl.ds(row0, stripe)],
                            out_ref.at[t].at[pl.ds(row0, stripe)])

    mesh = plsc.VectorSubcoreMesh(core_axis_name="c", subcore_axis_name="s")
    return pl.kernel(
        body,
        out_type=jax.ShapeDtypeStruct((n_tasks, n_pad, w), jnp.float32),
        mesh=mesh,
        scratch_types=[
            pltpu.VMEM((2, _K), jnp.int32),
            pltpu.VMEM((2, _K), jnp.int32),
            pltpu.VMEM((_K, w), jnp.float32),
            pltpu.VMEM((_K, w), jnp.float32),
            pltpu.VMEM_SHARED((n_pad, w), jnp.float32),
            pltpu.SemaphoreType.DMA,
            pltpu.SemaphoreType.DMA,
            pltpu.SemaphoreType.DMA,
            pltpu.SemaphoreType.DMA,
        ],
    )(p_blocked, idx, zeros)


def _edge_index_chunks(src, dst, n):
    e = src.shape[0]
    per_tile = e // _NS
    c_total = -(-per_tile // _K)
    if c_total % 2:
        c_total += 1
    pad = c_total * _K - per_tile
    src_t = jnp.pad(src.reshape(_NS, per_tile), ((0, 0), (0, pad)))
    dst_t = jnp.pad(dst.reshape(_NS, per_tile), ((0, 0), (0, pad)),
                    constant_values=n)
    idx = jnp.stack([src_t.reshape(_NS, c_total, _K),
                     dst_t.reshape(_NS, c_total, _K)], axis=2)
    dummy = jnp.zeros((_NS, 2, 2, _K), jnp.int32)
    dummy = dummy.at[:, :, 1, :].set(n)
    return jnp.concatenate([idx, dummy], axis=1), c_total


def kernel(x, edge_index, W1_rel, b1_rel, W1_root, W2_rel, b2_rel, W2_root,
           W3_rel, b3_rel, W3_root, W_pos, b_pos, W_rot, b_rot):
    n = x.shape[0]
    src, dst = edge_index[0], edge_index[1]
    stripe = -(-n // (_NS * 8)) * 8
    n_pad = stripe * _NS
    idx, c_total = _edge_index_chunks(src, dst, n)
    zeros = jnp.zeros((stripe, 128), jnp.float32)

    wcat1 = jnp.concatenate([W1_rel, W1_root], axis=1)
    o1 = _mm_first(x, wcat1, 128)
    agg1 = _sc_segsum_call(o1[:4], idx, zeros, n_pad, 1, c_total, 4)
    r1 = o1[4:]

    wcat2 = jnp.concatenate([W2_rel, W2_root], axis=1)
    o2 = _mm_mid(agg1, r1, b1_rel, wcat2, 128)
    agg2 = _sc_segsum_call(o2[:1], idx, zeros, n_pad, 2, c_total // 2, 2)

    wcat3 = jnp.concatenate([W3_rel, W3_root], axis=1)
    o3 = _mm_mid(agg2, o2[1:], b2_rel, wcat3, 128, partial=True)
    agg3 = _sc_segsum_call(o3, idx, zeros, n_pad, 2, c_total // 2, 2)

    w_head = jnp.concatenate([W_pos, W_rot], axis=1)
    b_head = jnp.concatenate([b_pos, b_rot])
    return _head(agg3, o3, b3_rel, w_head, b_head, 64)[:, :7]
